# Initial kernel scaffold; baseline (speedup 1.0000x reference)
import jax
import jax.numpy as jnp
from jax.experimental import pallas as pl


def kernel(x, edge_index, batch, W1, b1, W2, b2, W3, b3):
    raise NotImplementedError("write your pallas kernel here")


# speedup vs baseline: 13.1711x; 13.1711x over previous
import functools

import jax
import jax.numpy as jnp
from jax import lax
from jax.experimental import pallas as pl
from jax.experimental.pallas import tpu as pltpu
from jax.experimental.pallas import tpu_sc as plsc

N = 10000
E = 320000
DF = 128
NCLS = 10
NG = 64
DP = 16

NC = 2
NS = 16
NW = NC * NS
C = 80
EPT = E // NW
CPT = EPT // C
RPT = 624
RTAIL = N - RPT * NS

_HI = jax.lax.Precision.HIGHEST


@functools.lru_cache(maxsize=None)
def _make_scatter(D):
    mesh = plsc.VectorSubcoreMesh(core_axis_name="c", subcore_axis_name="s")

    @functools.partial(
        pl.kernel,
        out_type=jax.ShapeDtypeStruct((NC, N, D), jnp.float32),
        mesh=mesh,
        scratch_types=[
            pltpu.VMEM((CPT, C), jnp.int32),
            pltpu.VMEM((CPT, C), jnp.int32),
            pltpu.VMEM((C, D), jnp.float32),
            pltpu.VMEM_SHARED((N, D), jnp.float32),
            pltpu.SemaphoreType.DMA,
        ],
    )
    def scat(g_hbm, src_hbm, dst_hbm, zeros_hbm, out_hbm, srcv, dstv, rows, acc, sem):
        cid = lax.axis_index("c")
        sid = lax.axis_index("s")
        wid = cid * NS + sid
        pltpu.sync_copy(zeros_hbm.at[pl.ds(sid * RPT, RPT)],
                        acc.at[pl.ds(sid * RPT, RPT)])

        @pl.when(sid == NS - 1)
        def _():
            pltpu.sync_copy(zeros_hbm.at[pl.ds(NS * RPT, RTAIL)],
                            acc.at[pl.ds(NS * RPT, RTAIL)])

        pltpu.sync_copy(src_hbm.at[wid], srcv)
        pltpu.sync_copy(dst_hbm.at[wid], dstv)
        plsc.subcore_barrier()

        def body(j, carry):
            pltpu.async_copy(g_hbm.at[srcv.at[j]], rows, sem).wait()
            pltpu.sync_copy(rows, acc.at[dstv.at[j]], add=True)
            return carry

        lax.fori_loop(0, CPT, body, 0)
        plsc.subcore_barrier()
        pltpu.sync_copy(acc.at[pl.ds(sid * RPT, RPT)],
                        out_hbm.at[cid, pl.ds(sid * RPT, RPT)])

        @pl.when(sid == NS - 1)
        def _():
            pltpu.sync_copy(acc.at[pl.ds(NS * RPT, RTAIL)],
                            out_hbm.at[cid, pl.ds(NS * RPT, RTAIL)])

    return scat


def _scat(*args):
    return _make_scatter(DF)(*args)


BLK = 2000


def _tcA_body(x_ref, w_ref, degp_ref, g_ref, dis_ref):
    deg = degp_ref[0, :, 0:1] + degp_ref[1, :, 0:1] + 1.0
    dis = lax.rsqrt(deg)
    h = jnp.dot(x_ref[...], w_ref[...], preferred_element_type=jnp.float32,
                precision=_HI)
    g_ref[...] = h * dis
    dis_ref[...] = dis


def _tcA(x, W1, degp):
    return pl.pallas_call(
        _tcA_body,
        grid=(N // BLK,),
        in_specs=[
            pl.BlockSpec((BLK, DF), lambda i: (i, 0)),
            pl.BlockSpec((DF, DF), lambda i: (0, 0)),
            pl.BlockSpec((NC, BLK, DF), lambda i: (0, i, 0)),
        ],
        out_specs=[
            pl.BlockSpec((BLK, DF), lambda i: (i, 0)),
            pl.BlockSpec((BLK, 1), lambda i: (i, 0)),
        ],
        out_shape=[
            jax.ShapeDtypeStruct((N, DF), jnp.float32),
            jax.ShapeDtypeStruct((N, 1), jnp.float32),
        ],
    )(x, W1, degp)


def _tcB_body(sp_ref, g_ref, dis_ref, b_ref, w_ref, out_ref):
    dis = dis_ref[...]
    pre = dis * (sp_ref[0] + sp_ref[1] + g_ref[...]) + b_ref[...]
    h = jnp.maximum(pre, 0.0)
    out_ref[...] = dis * jnp.dot(h, w_ref[...], preferred_element_type=jnp.float32,
                                 precision=_HI)


def _tcB(sp, g, dis, b2d, W):
    return pl.pallas_call(
        _tcB_body,
        grid=(N // BLK,),
        in_specs=[
            pl.BlockSpec((NC, BLK, DF), lambda i: (0, i, 0)),
            pl.BlockSpec((BLK, DF), lambda i: (i, 0)),
            pl.BlockSpec((BLK, 1), lambda i: (i, 0)),
            pl.BlockSpec((1, DF), lambda i: (0, 0)),
            pl.BlockSpec((DF, DF), lambda i: (0, 0)),
        ],
        out_specs=pl.BlockSpec((BLK, DF), lambda i: (i, 0)),
        out_shape=jax.ShapeDtypeStruct((N, DF), jnp.float32),
    )(sp, g, dis, b2d, W)


def _tcC_body(sp_ref, g_ref, dis_ref, b_ref, out_ref):
    dis = dis_ref[...]
    pre = dis * (sp_ref[0] + sp_ref[1] + g_ref[...]) + b_ref[...]
    out_ref[...] = dis * jnp.maximum(pre, 0.0)


def _tcC(sp, g, dis, b2d):
    return pl.pallas_call(
        _tcC_body,
        grid=(N // BLK,),
        in_specs=[
            pl.BlockSpec((NC, BLK, DF), lambda i: (0, i, 0)),
            pl.BlockSpec((BLK, DF), lambda i: (i, 0)),
            pl.BlockSpec((BLK, 1), lambda i: (i, 0)),
            pl.BlockSpec((1, DF), lambda i: (0, 0)),
        ],
        out_specs=pl.BlockSpec((BLK, DF), lambda i: (i, 0)),
        out_shape=jax.ShapeDtypeStruct((N, DF), jnp.float32),
    )(sp, g, dis, b2d)


def _tcD_body(sp_ref, u_ref, dis_ref, w_ref, b_ref, batch_ref, out_ref):
    t = sp_ref[0] + sp_ref[1] + u_ref[...]
    h = dis_ref[...] * jnp.dot(t, w_ref[...], preferred_element_type=jnp.float32,
                               precision=_HI) + b_ref[...]
    ids = lax.broadcasted_iota(jnp.int32, (NG, N), 0)
    m = (ids == batch_ref[...]).astype(jnp.float32)
    sums = jnp.dot(m, h, preferred_element_type=jnp.float32, precision=_HI)
    counts = jnp.sum(m, axis=1, keepdims=True)
    pooled = sums / jnp.maximum(counts, 1.0)
    logits = pooled[:, :NCLS]
    mx = jnp.max(logits, axis=1, keepdims=True)
    s = logits - mx
    lse = jnp.log(jnp.sum(jnp.exp(s), axis=1, keepdims=True))
    out_ref[...] = s - lse


def _tcD(sp, u, dis, W3p, b2d, batch2d):
    return pl.pallas_call(
        _tcD_body,
        in_specs=[
            pl.BlockSpec((NC, N, DF), lambda: (0, 0, 0)),
            pl.BlockSpec((N, DF), lambda: (0, 0)),
            pl.BlockSpec((N, 1), lambda: (0, 0)),
            pl.BlockSpec((DF, DP), lambda: (0, 0)),
            pl.BlockSpec((1, DP), lambda: (0, 0)),
            pl.BlockSpec((1, N), lambda: (0, 0)),
        ],
        out_specs=pl.BlockSpec((NG, NCLS), lambda: (0, 0)),
        out_shape=jax.ShapeDtypeStruct((NG, NCLS), jnp.float32),
    )(sp, u, dis, W3p, b2d, batch2d)


def kernel(x, edge_index, batch, W1, b1, W2, b2, W3, b3):
    src3 = edge_index[0].reshape(NW, CPT, C)
    dst3 = edge_index[1].reshape(NW, CPT, C)
    ones128 = jnp.concatenate(
        [jnp.ones((N, 1), jnp.float32), jnp.zeros((N, DF - 1), jnp.float32)], axis=1)
    z128 = jnp.zeros((N, DF), jnp.float32)

    degp = _scat(ones128, src3, dst3, z128)
    g1, dis = _tcA(x, W1, degp)
    s1 = _scat(g1, src3, dst3, z128)
    g2 = _tcB(s1, g1, dis, b1.reshape(1, DF), W2)
    s2 = _scat(g2, src3, dst3, z128)
    u = _tcC(s2, g2, dis, b2.reshape(1, DF))
    s3 = _scat(u, src3, dst3, z128)
    W3p = jnp.zeros((DF, DP), jnp.float32).at[:, :NCLS].set(W3)
    b3p = jnp.zeros((1, DP), jnp.float32).at[:, :NCLS].set(b3.reshape(1, NCLS))
    return _tcD(s3, u, dis, W3p, b3p, batch.reshape(1, N))


# speedup vs baseline: 16.9058x; 1.2835x over previous
import functools

import jax
import jax.numpy as jnp
from jax import lax
from jax.experimental import pallas as pl
from jax.experimental.pallas import tpu as pltpu
from jax.experimental.pallas import tpu_sc as plsc

N = 10000
E = 320000
DF = 128
NCLS = 10
NG = 64
DP = 16

NC = 2
NS = 16
NW = NC * NS
C = 80
EPT = E // NW
CPT = EPT // C
RPT = 624
RTAIL = N - RPT * NS

_HI = jax.lax.Precision.HIGHEST


@functools.lru_cache(maxsize=None)
def _make_scatter(D):
    mesh = plsc.VectorSubcoreMesh(core_axis_name="c", subcore_axis_name="s")

    @functools.partial(
        pl.kernel,
        out_type=jax.ShapeDtypeStruct((NC, N, D), jnp.float32),
        mesh=mesh,
        scratch_types=[
            pltpu.VMEM((EPT,), jnp.int32),
            pltpu.VMEM((CPT, C), jnp.int32),
            pltpu.VMEM((C, D), jnp.float32),
            pltpu.VMEM((C, D), jnp.float32),
            pltpu.VMEM_SHARED((N, D), jnp.float32),
            pltpu.SemaphoreType.DMA,
            pltpu.SemaphoreType.DMA,
        ],
    )
    def scat(g_hbm, src_hbm, dst_hbm, zeros_hbm, out_hbm, srcv, dstv,
             rows0, rows1, acc, sem0, sem1):
        cid = lax.axis_index("c")
        sid = lax.axis_index("s")
        wid = cid * NS + sid
        pltpu.sync_copy(zeros_hbm.at[pl.ds(sid * RPT, RPT)],
                        acc.at[pl.ds(sid * RPT, RPT)])

        @pl.when(sid == NS - 1)
        def _():
            pltpu.sync_copy(zeros_hbm.at[pl.ds(NS * RPT, RTAIL)],
                            acc.at[pl.ds(NS * RPT, RTAIL)])

        pltpu.sync_copy(src_hbm.at[pl.ds(wid * EPT, EPT)], srcv)
        pltpu.sync_copy(dst_hbm.at[wid], dstv)
        plsc.subcore_barrier()

        def sidx(j):
            return srcv.at[pl.ds(pl.multiple_of(j * C, C), C)]

        pltpu.async_copy(g_hbm.at[sidx(0)], rows0, sem0)

        def body(t, carry):
            a = 2 * t
            pltpu.make_async_copy(g_hbm.at[sidx(a)], rows0, sem0).wait()
            pltpu.async_copy(g_hbm.at[sidx(a + 1)], rows1, sem1)
            pltpu.sync_copy(rows0, acc.at[dstv.at[a]], add=True)
            pltpu.make_async_copy(g_hbm.at[sidx(a + 1)], rows1, sem1).wait()
            pltpu.async_copy(g_hbm.at[sidx(a + 2)], rows0, sem0)
            pltpu.sync_copy(rows1, acc.at[dstv.at[a + 1]], add=True)
            return carry

        lax.fori_loop(0, (CPT - 1) // 2, body, 0)
        pltpu.make_async_copy(g_hbm.at[sidx(CPT - 1)], rows0, sem0).wait()
        pltpu.sync_copy(rows0, acc.at[dstv.at[CPT - 1]], add=True)
        plsc.subcore_barrier()
        pltpu.sync_copy(acc.at[pl.ds(sid * RPT, RPT)],
                        out_hbm.at[cid, pl.ds(sid * RPT, RPT)])

        @pl.when(sid == NS - 1)
        def _():
            pltpu.sync_copy(acc.at[pl.ds(NS * RPT, RTAIL)],
                            out_hbm.at[cid, pl.ds(NS * RPT, RTAIL)])

    return scat


def _scat(*args):
    return _make_scatter(DF)(*args)


BLK = 2000


def _tcA_body(x_ref, w_ref, degp_ref, g_ref, dis_ref):
    deg = degp_ref[0, :, 0:1] + degp_ref[1, :, 0:1] + 1.0
    dis = lax.rsqrt(deg)
    h = jnp.dot(x_ref[...], w_ref[...], preferred_element_type=jnp.float32,
                precision=_HI)
    g_ref[...] = h * dis
    dis_ref[...] = dis


def _tcA(x, W1, degp):
    return pl.pallas_call(
        _tcA_body,
        grid=(N // BLK,),
        in_specs=[
            pl.BlockSpec((BLK, DF), lambda i: (i, 0)),
            pl.BlockSpec((DF, DF), lambda i: (0, 0)),
            pl.BlockSpec((NC, BLK, DF), lambda i: (0, i, 0)),
        ],
        out_specs=[
            pl.BlockSpec((BLK, DF), lambda i: (i, 0)),
            pl.BlockSpec((BLK, 1), lambda i: (i, 0)),
        ],
        out_shape=[
            jax.ShapeDtypeStruct((N, DF), jnp.float32),
            jax.ShapeDtypeStruct((N, 1), jnp.float32),
        ],
    )(x, W1, degp)


def _tcB_body(sp_ref, g_ref, dis_ref, b_ref, w_ref, out_ref):
    dis = dis_ref[...]
    pre = dis * (sp_ref[0] + sp_ref[1] + g_ref[...]) + b_ref[...]
    h = jnp.maximum(pre, 0.0)
    out_ref[...] = dis * jnp.dot(h, w_ref[...], preferred_element_type=jnp.float32,
                                 precision=_HI)


def _tcB(sp, g, dis, b2d, W):
    return pl.pallas_call(
        _tcB_body,
        grid=(N // BLK,),
        in_specs=[
            pl.BlockSpec((NC, BLK, DF), lambda i: (0, i, 0)),
            pl.BlockSpec((BLK, DF), lambda i: (i, 0)),
            pl.BlockSpec((BLK, 1), lambda i: (i, 0)),
            pl.BlockSpec((1, DF), lambda i: (0, 0)),
            pl.BlockSpec((DF, DF), lambda i: (0, 0)),
        ],
        out_specs=pl.BlockSpec((BLK, DF), lambda i: (i, 0)),
        out_shape=jax.ShapeDtypeStruct((N, DF), jnp.float32),
    )(sp, g, dis, b2d, W)


def _tcC_body(sp_ref, g_ref, dis_ref, b_ref, out_ref):
    dis = dis_ref[...]
    pre = dis * (sp_ref[0] + sp_ref[1] + g_ref[...]) + b_ref[...]
    out_ref[...] = dis * jnp.maximum(pre, 0.0)


def _tcC(sp, g, dis, b2d):
    return pl.pallas_call(
        _tcC_body,
        grid=(N // BLK,),
        in_specs=[
            pl.BlockSpec((NC, BLK, DF), lambda i: (0, i, 0)),
            pl.BlockSpec((BLK, DF), lambda i: (i, 0)),
            pl.BlockSpec((BLK, 1), lambda i: (i, 0)),
            pl.BlockSpec((1, DF), lambda i: (0, 0)),
        ],
        out_specs=pl.BlockSpec((BLK, DF), lambda i: (i, 0)),
        out_shape=jax.ShapeDtypeStruct((N, DF), jnp.float32),
    )(sp, g, dis, b2d)


def _tcD_body(sp_ref, u_ref, dis_ref, w_ref, b_ref, batch_ref, out_ref):
    t = sp_ref[0] + sp_ref[1] + u_ref[...]
    h = dis_ref[...] * jnp.dot(t, w_ref[...], preferred_element_type=jnp.float32,
                               precision=_HI) + b_ref[...]
    ids = lax.broadcasted_iota(jnp.int32, (NG, N), 0)
    m = (ids == batch_ref[...]).astype(jnp.float32)
    sums = jnp.dot(m, h, preferred_element_type=jnp.float32, precision=_HI)
    counts = jnp.sum(m, axis=1, keepdims=True)
    pooled = sums / jnp.maximum(counts, 1.0)
    logits = pooled[:, :NCLS]
    mx = jnp.max(logits, axis=1, keepdims=True)
    s = logits - mx
    lse = jnp.log(jnp.sum(jnp.exp(s), axis=1, keepdims=True))
    out_ref[...] = s - lse


def _tcD(sp, u, dis, W3p, b2d, batch2d):
    return pl.pallas_call(
        _tcD_body,
        in_specs=[
            pl.BlockSpec((NC, N, DF), lambda: (0, 0, 0)),
            pl.BlockSpec((N, DF), lambda: (0, 0)),
            pl.BlockSpec((N, 1), lambda: (0, 0)),
            pl.BlockSpec((DF, DP), lambda: (0, 0)),
            pl.BlockSpec((1, DP), lambda: (0, 0)),
            pl.BlockSpec((1, N), lambda: (0, 0)),
        ],
        out_specs=pl.BlockSpec((NG, NCLS), lambda: (0, 0)),
        out_shape=jax.ShapeDtypeStruct((NG, NCLS), jnp.float32),
    )(sp, u, dis, W3p, b2d, batch2d)


def kernel(x, edge_index, batch, W1, b1, W2, b2, W3, b3):
    src3 = edge_index[0]
    dst3 = edge_index[1].reshape(NW, CPT, C)
    ones128 = jnp.concatenate(
        [jnp.ones((N, 1), jnp.float32), jnp.zeros((N, DF - 1), jnp.float32)], axis=1)
    z128 = jnp.zeros((N, DF), jnp.float32)

    degp = _scat(ones128, src3, dst3, z128)
    g1, dis = _tcA(x, W1, degp)
    s1 = _scat(g1, src3, dst3, z128)
    g2 = _tcB(s1, g1, dis, b1.reshape(1, DF), W2)
    s2 = _scat(g2, src3, dst3, z128)
    u = _tcC(s2, g2, dis, b2.reshape(1, DF))
    s3 = _scat(u, src3, dst3, z128)
    W3p = jnp.zeros((DF, DP), jnp.float32).at[:, :NCLS].set(W3)
    b3p = jnp.zeros((1, DP), jnp.float32).at[:, :NCLS].set(b3.reshape(1, NCLS))
    return _tcD(s3, u, dis, W3p, b3p, batch.reshape(1, N))


# speedup vs baseline: 21.1563x; 1.2514x over previous
import functools

import jax
import jax.numpy as jnp
from jax import lax
from jax.experimental import pallas as pl
from jax.experimental.pallas import tpu as pltpu
from jax.experimental.pallas import tpu_sc as plsc

N = 10000
E = 320000
DF = 128
NCLS = 10
NG = 64
DP = 16

NC = 2
NS = 16
NW = NC * NS
C = 80
EPT = E // NW
CPT = EPT // C
RPT = 624
RTAIL = N - RPT * NS

_HI = jax.lax.Precision.HIGHEST


@functools.lru_cache(maxsize=None)
def _make_scatter(D):
    mesh = plsc.VectorSubcoreMesh(core_axis_name="c", subcore_axis_name="s")

    @functools.partial(
        pl.kernel,
        out_type=jax.ShapeDtypeStruct((NC, N, D), jnp.float32),
        mesh=mesh,
        scratch_types=[
            pltpu.VMEM((EPT,), jnp.int32),
            pltpu.VMEM((CPT, C), jnp.int32),
            pltpu.VMEM((C, D), jnp.float32),
            pltpu.VMEM((C, D), jnp.float32),
            pltpu.VMEM_SHARED((N, D), jnp.float32),
            pltpu.SemaphoreType.DMA,
            pltpu.SemaphoreType.DMA,
        ],
    )
    def scat(g_hbm, src_hbm, dst_hbm, zeros_hbm, out_hbm, srcv, dstv,
             rows0, rows1, acc, sem0, sem1):
        cid = lax.axis_index("c")
        sid = lax.axis_index("s")
        wid = cid * NS + sid
        pltpu.sync_copy(zeros_hbm.at[pl.ds(sid * RPT, RPT)],
                        acc.at[pl.ds(sid * RPT, RPT)])

        @pl.when(sid == NS - 1)
        def _():
            pltpu.sync_copy(zeros_hbm.at[pl.ds(NS * RPT, RTAIL)],
                            acc.at[pl.ds(NS * RPT, RTAIL)])

        pltpu.sync_copy(src_hbm.at[pl.ds(wid * EPT, EPT)], srcv)
        pltpu.sync_copy(dst_hbm.at[wid], dstv)
        plsc.subcore_barrier()

        def sidx(j):
            return srcv.at[pl.ds(pl.multiple_of(j * C, C), C)]

        pltpu.async_copy(g_hbm.at[sidx(0)], rows0, sem0)

        def body(t, carry):
            a = 2 * t
            pltpu.make_async_copy(g_hbm.at[sidx(a)], rows0, sem0).wait()
            pltpu.async_copy(g_hbm.at[sidx(a + 1)], rows1, sem1)
            pltpu.sync_copy(rows0, acc.at[dstv.at[a]], add=True)
            pltpu.make_async_copy(g_hbm.at[sidx(a + 1)], rows1, sem1).wait()
            pltpu.async_copy(g_hbm.at[sidx(a + 2)], rows0, sem0)
            pltpu.sync_copy(rows1, acc.at[dstv.at[a + 1]], add=True)
            return carry

        lax.fori_loop(0, (CPT - 1) // 2, body, 0)
        pltpu.make_async_copy(g_hbm.at[sidx(CPT - 1)], rows0, sem0).wait()
        pltpu.sync_copy(rows0, acc.at[dstv.at[CPT - 1]], add=True)
        plsc.subcore_barrier()
        pltpu.sync_copy(acc.at[pl.ds(sid * RPT, RPT)],
                        out_hbm.at[cid, pl.ds(sid * RPT, RPT)])

        @pl.when(sid == NS - 1)
        def _():
            pltpu.sync_copy(acc.at[pl.ds(NS * RPT, RTAIL)],
                            out_hbm.at[cid, pl.ds(NS * RPT, RTAIL)])

    return scat


def _scat(*args):
    return _make_scatter(DF)(*args)


NPAD = 10112


@functools.lru_cache(maxsize=None)
def _make_deg():
    mesh = plsc.VectorSubcoreMesh(core_axis_name="c", subcore_axis_name="s")

    @functools.partial(
        pl.kernel,
        out_type=jax.ShapeDtypeStruct((NW, NPAD), jnp.float32),
        mesh=mesh,
        compiler_params=pltpu.CompilerParams(needs_layout_passes=False),
        scratch_types=[
            pltpu.VMEM((EPT,), jnp.int32),
            pltpu.VMEM((NPAD,), jnp.float32),
        ],
    )
    def degk(dst_hbm, zeros_hbm, out_hbm, dstv, degloc):
        cid = lax.axis_index("c")
        sid = lax.axis_index("s")
        wid = cid * NS + sid
        pltpu.sync_copy(zeros_hbm, degloc)
        pltpu.sync_copy(dst_hbm.at[pl.ds(wid * EPT, EPT)], dstv)

        def body(i, carry):
            idx = dstv[pl.ds(i * 16, 16)]
            plsc.addupdate_scatter(degloc, [idx], jnp.ones((16,), jnp.float32))
            return carry

        lax.fori_loop(0, EPT // 16, body, 0)
        pltpu.sync_copy(degloc, out_hbm.at[wid])

    return degk


BLK = 2000


def _tcdis_body(degp_ref, dis_ref):
    ones_col = jnp.ones((NW, 1), jnp.float32)
    deg = lax.dot_general(degp_ref[...], ones_col, (((0,), (0,)), ((), ())),
                          precision=_HI, preferred_element_type=jnp.float32)
    dis_ref[...] = lax.rsqrt(deg[:N] + 1.0)


def _tcdis(degp):
    return pl.pallas_call(
        _tcdis_body,
        in_specs=[pl.BlockSpec((NW, NPAD), lambda: (0, 0))],
        out_specs=pl.BlockSpec((N, 1), lambda: (0, 0)),
        out_shape=jax.ShapeDtypeStruct((N, 1), jnp.float32),
    )(degp)


def _tcA_body(x_ref, w_ref, dis_ref, g_ref):
    h = jnp.dot(x_ref[...], w_ref[...], preferred_element_type=jnp.float32,
                precision=_HI)
    g_ref[...] = h * dis_ref[...]


def _tcA(x, W1, dis):
    return pl.pallas_call(
        _tcA_body,
        grid=(N // BLK,),
        in_specs=[
            pl.BlockSpec((BLK, DF), lambda i: (i, 0)),
            pl.BlockSpec((DF, DF), lambda i: (0, 0)),
            pl.BlockSpec((BLK, 1), lambda i: (i, 0)),
        ],
        out_specs=pl.BlockSpec((BLK, DF), lambda i: (i, 0)),
        out_shape=jax.ShapeDtypeStruct((N, DF), jnp.float32),
    )(x, W1, dis)


def _tcB_body(sp_ref, g_ref, dis_ref, b_ref, w_ref, out_ref):
    dis = dis_ref[...]
    pre = dis * (sp_ref[0] + sp_ref[1] + g_ref[...]) + b_ref[...]
    h = jnp.maximum(pre, 0.0)
    out_ref[...] = dis * jnp.dot(h, w_ref[...], preferred_element_type=jnp.float32,
                                 precision=_HI)


def _tcB(sp, g, dis, b2d, W):
    return pl.pallas_call(
        _tcB_body,
        grid=(N // BLK,),
        in_specs=[
            pl.BlockSpec((NC, BLK, DF), lambda i: (0, i, 0)),
            pl.BlockSpec((BLK, DF), lambda i: (i, 0)),
            pl.BlockSpec((BLK, 1), lambda i: (i, 0)),
            pl.BlockSpec((1, DF), lambda i: (0, 0)),
            pl.BlockSpec((DF, DF), lambda i: (0, 0)),
        ],
        out_specs=pl.BlockSpec((BLK, DF), lambda i: (i, 0)),
        out_shape=jax.ShapeDtypeStruct((N, DF), jnp.float32),
    )(sp, g, dis, b2d, W)


def _tcC_body(sp_ref, g_ref, dis_ref, b_ref, out_ref):
    dis = dis_ref[...]
    pre = dis * (sp_ref[0] + sp_ref[1] + g_ref[...]) + b_ref[...]
    out_ref[...] = dis * jnp.maximum(pre, 0.0)


def _tcC(sp, g, dis, b2d):
    return pl.pallas_call(
        _tcC_body,
        grid=(N // BLK,),
        in_specs=[
            pl.BlockSpec((NC, BLK, DF), lambda i: (0, i, 0)),
            pl.BlockSpec((BLK, DF), lambda i: (i, 0)),
            pl.BlockSpec((BLK, 1), lambda i: (i, 0)),
            pl.BlockSpec((1, DF), lambda i: (0, 0)),
        ],
        out_specs=pl.BlockSpec((BLK, DF), lambda i: (i, 0)),
        out_shape=jax.ShapeDtypeStruct((N, DF), jnp.float32),
    )(sp, g, dis, b2d)


def _tcD_body(sp_ref, u_ref, dis_ref, w_ref, b_ref, batch_ref, out_ref):
    t = sp_ref[0] + sp_ref[1] + u_ref[...]
    h = dis_ref[...] * jnp.dot(t, w_ref[...], preferred_element_type=jnp.float32,
                               precision=_HI) + b_ref[...]
    ids = lax.broadcasted_iota(jnp.int32, (NG, N), 0)
    m = (ids == batch_ref[...]).astype(jnp.float32)
    sums = jnp.dot(m, h, preferred_element_type=jnp.float32, precision=_HI)
    counts = jnp.sum(m, axis=1, keepdims=True)
    pooled = sums / jnp.maximum(counts, 1.0)
    logits = pooled[:, :NCLS]
    mx = jnp.max(logits, axis=1, keepdims=True)
    s = logits - mx
    lse = jnp.log(jnp.sum(jnp.exp(s), axis=1, keepdims=True))
    out_ref[...] = s - lse


def _tcD(sp, u, dis, W3p, b2d, batch2d):
    return pl.pallas_call(
        _tcD_body,
        in_specs=[
            pl.BlockSpec((NC, N, DF), lambda: (0, 0, 0)),
            pl.BlockSpec((N, DF), lambda: (0, 0)),
            pl.BlockSpec((N, 1), lambda: (0, 0)),
            pl.BlockSpec((DF, DP), lambda: (0, 0)),
            pl.BlockSpec((1, DP), lambda: (0, 0)),
            pl.BlockSpec((1, N), lambda: (0, 0)),
        ],
        out_specs=pl.BlockSpec((NG, NCLS), lambda: (0, 0)),
        out_shape=jax.ShapeDtypeStruct((NG, NCLS), jnp.float32),
    )(sp, u, dis, W3p, b2d, batch2d)


def kernel(x, edge_index, batch, W1, b1, W2, b2, W3, b3):
    src3 = edge_index[0]
    dst3 = edge_index[1].reshape(NW, CPT, C)
    z128 = jnp.zeros((N, DF), jnp.float32)

    degp = _make_deg()(edge_index[1], jnp.zeros((NPAD,), jnp.float32))
    dis = _tcdis(degp)
    g1 = _tcA(x, W1, dis)
    s1 = _scat(g1, src3, dst3, z128)
    g2 = _tcB(s1, g1, dis, b1.reshape(1, DF), W2)
    s2 = _scat(g2, src3, dst3, z128)
    u = _tcC(s2, g2, dis, b2.reshape(1, DF))
    s3 = _scat(u, src3, dst3, z128)
    W3p = jnp.zeros((DF, DP), jnp.float32).at[:, :NCLS].set(W3)
    b3p = jnp.zeros((1, DP), jnp.float32).at[:, :NCLS].set(b3.reshape(1, NCLS))
    return _tcD(s3, u, dis, W3p, b3p, batch.reshape(1, N))


# speedup vs baseline: 30.0273x; 1.4193x over previous
import functools

import jax
import jax.numpy as jnp
from jax import lax
from jax.experimental import pallas as pl
from jax.experimental.pallas import tpu as pltpu
from jax.experimental.pallas import tpu_sc as plsc

N = 10000
E = 320000
DF = 128
NCLS = 10
NG = 64
DP = 16

NC = 2
NS = 16
NW = NC * NS
C = 80
EPT = E // NW
CPT = EPT // C
RPT = 624
RTAIL = N - RPT * NS

_HI = jax.lax.Precision.HIGHEST


@functools.lru_cache(maxsize=None)
def _make_scatter(D):
    mesh = plsc.VectorSubcoreMesh(core_axis_name="c", subcore_axis_name="s")

    @functools.partial(
        pl.kernel,
        out_type=jax.ShapeDtypeStruct((NC, N, D), jnp.float32),
        mesh=mesh,
        compiler_params=pltpu.CompilerParams(needs_layout_passes=False),
        scratch_types=[
            pltpu.VMEM((EPT,), jnp.int32),
            pltpu.VMEM((EPT,), jnp.int32),
            pltpu.VMEM((C, D), jnp.float32),
            pltpu.VMEM((C, D), jnp.float32),
            pltpu.VMEM((C, D), jnp.float32),
            pltpu.VMEM_SHARED((N, D), jnp.float32),
            pltpu.SemaphoreType.DMA,
            pltpu.SemaphoreType.DMA,
            pltpu.SemaphoreType.DMA,
        ],
    )
    def scat(g_hbm, src_hbm, dst_hbm, zeros_hbm, out_hbm, srcv, dstv,
             rows0, rows1, rows2, acc, sem0, sem1, sem2):
        cid = lax.axis_index("c")
        sid = lax.axis_index("s")
        wid = cid * NS + sid
        pltpu.sync_copy(zeros_hbm.at[pl.ds(sid * RPT, RPT)],
                        acc.at[pl.ds(sid * RPT, RPT)])

        @pl.when(sid == NS - 1)
        def _():
            pltpu.sync_copy(zeros_hbm.at[pl.ds(NS * RPT, RTAIL)],
                            acc.at[pl.ds(NS * RPT, RTAIL)])

        pltpu.sync_copy(src_hbm.at[pl.ds(wid * EPT, EPT)], srcv)
        pltpu.sync_copy(dst_hbm.at[pl.ds(wid * EPT, EPT)], dstv)
        plsc.subcore_barrier()

        def sidx(j):
            return srcv.at[pl.ds(pl.multiple_of(j * C, C), C)]

        def didx(j):
            return dstv.at[pl.ds(pl.multiple_of(j * C, C), C)]

        pltpu.async_copy(g_hbm.at[sidx(0)], rows0, sem0)
        pltpu.async_copy(g_hbm.at[sidx(1)], rows1, sem1)

        def body(t, carry):
            a = 3 * t
            pltpu.make_async_copy(g_hbm.at[sidx(a)], rows0, sem0).wait()
            pltpu.async_copy(g_hbm.at[sidx(a + 2)], rows2, sem2)
            pltpu.sync_copy(rows0, acc.at[didx(a)], add=True)
            pltpu.make_async_copy(g_hbm.at[sidx(a + 1)], rows1, sem1).wait()
            pltpu.async_copy(g_hbm.at[sidx(a + 3)], rows0, sem0)
            pltpu.sync_copy(rows1, acc.at[didx(a + 1)], add=True)
            pltpu.make_async_copy(g_hbm.at[sidx(a + 2)], rows2, sem2).wait()
            pltpu.async_copy(g_hbm.at[sidx(a + 4)], rows1, sem1)
            pltpu.sync_copy(rows2, acc.at[didx(a + 2)], add=True)
            return carry

        lax.fori_loop(0, (CPT - 2) // 3, body, 0)
        pltpu.make_async_copy(g_hbm.at[sidx(CPT - 2)], rows0, sem0).wait()
        pltpu.sync_copy(rows0, acc.at[didx(CPT - 2)], add=True)
        pltpu.make_async_copy(g_hbm.at[sidx(CPT - 1)], rows1, sem1).wait()
        pltpu.sync_copy(rows1, acc.at[didx(CPT - 1)], add=True)
        plsc.subcore_barrier()
        pltpu.sync_copy(acc.at[pl.ds(sid * RPT, RPT)],
                        out_hbm.at[cid, pl.ds(sid * RPT, RPT)])

        @pl.when(sid == NS - 1)
        def _():
            pltpu.sync_copy(acc.at[pl.ds(NS * RPT, RTAIL)],
                            out_hbm.at[cid, pl.ds(NS * RPT, RTAIL)])

    return scat


def _scat(*args):
    return _make_scatter(DF)(*args)


NPAD = 10112


@functools.lru_cache(maxsize=None)
def _make_deg():
    mesh = plsc.VectorSubcoreMesh(core_axis_name="c", subcore_axis_name="s")

    @functools.partial(
        pl.kernel,
        out_type=jax.ShapeDtypeStruct((NW, NPAD), jnp.float32),
        mesh=mesh,
        compiler_params=pltpu.CompilerParams(needs_layout_passes=False),
        scratch_types=[
            pltpu.VMEM((EPT,), jnp.int32),
            pltpu.VMEM((NPAD,), jnp.float32),
        ],
    )
    def degk(dst_hbm, zeros_hbm, out_hbm, dstv, degloc):
        cid = lax.axis_index("c")
        sid = lax.axis_index("s")
        wid = cid * NS + sid
        pltpu.sync_copy(zeros_hbm, degloc)
        pltpu.sync_copy(dst_hbm.at[pl.ds(wid * EPT, EPT)], dstv)

        def body(i, carry):
            idx = dstv[pl.ds(i * 16, 16)]
            plsc.addupdate_scatter(degloc, [idx], jnp.ones((16,), jnp.float32))
            return carry

        lax.fori_loop(0, EPT // 16, body, 0)
        pltpu.sync_copy(degloc, out_hbm.at[wid])

    return degk


BLK = 2000


def _tcdis_body(degp_ref, dis_ref):
    ones_col = jnp.ones((NW, 1), jnp.float32)
    deg = lax.dot_general(degp_ref[...], ones_col, (((0,), (0,)), ((), ())),
                          precision=_HI, preferred_element_type=jnp.float32)
    dis_ref[...] = lax.rsqrt(deg[:N] + 1.0)


def _tcdis(degp):
    return pl.pallas_call(
        _tcdis_body,
        in_specs=[pl.BlockSpec((NW, NPAD), lambda: (0, 0))],
        out_specs=pl.BlockSpec((N, 1), lambda: (0, 0)),
        out_shape=jax.ShapeDtypeStruct((N, 1), jnp.float32),
    )(degp)


def _tcA_body(x_ref, w_ref, dis_ref, g_ref):
    h = jnp.dot(x_ref[...], w_ref[...], preferred_element_type=jnp.float32,
                precision=_HI)
    g_ref[...] = h * dis_ref[...]


def _tcA(x, W1, dis):
    return pl.pallas_call(
        _tcA_body,
        grid=(N // BLK,),
        in_specs=[
            pl.BlockSpec((BLK, DF), lambda i: (i, 0)),
            pl.BlockSpec((DF, DF), lambda i: (0, 0)),
            pl.BlockSpec((BLK, 1), lambda i: (i, 0)),
        ],
        out_specs=pl.BlockSpec((BLK, DF), lambda i: (i, 0)),
        out_shape=jax.ShapeDtypeStruct((N, DF), jnp.float32),
    )(x, W1, dis)


def _tcB_body(sp_ref, g_ref, dis_ref, b_ref, w_ref, out_ref):
    dis = dis_ref[...]
    pre = dis * (sp_ref[0] + sp_ref[1] + g_ref[...]) + b_ref[...]
    h = jnp.maximum(pre, 0.0)
    out_ref[...] = dis * jnp.dot(h, w_ref[...], preferred_element_type=jnp.float32,
                                 precision=_HI)


def _tcB(sp, g, dis, b2d, W):
    return pl.pallas_call(
        _tcB_body,
        grid=(N // BLK,),
        in_specs=[
            pl.BlockSpec((NC, BLK, DF), lambda i: (0, i, 0)),
            pl.BlockSpec((BLK, DF), lambda i: (i, 0)),
            pl.BlockSpec((BLK, 1), lambda i: (i, 0)),
            pl.BlockSpec((1, DF), lambda i: (0, 0)),
            pl.BlockSpec((DF, DF), lambda i: (0, 0)),
        ],
        out_specs=pl.BlockSpec((BLK, DF), lambda i: (i, 0)),
        out_shape=jax.ShapeDtypeStruct((N, DF), jnp.float32),
    )(sp, g, dis, b2d, W)


def _tcC_body(sp_ref, g_ref, dis_ref, b_ref, out_ref):
    dis = dis_ref[...]
    pre = dis * (sp_ref[0] + sp_ref[1] + g_ref[...]) + b_ref[...]
    out_ref[...] = dis * jnp.maximum(pre, 0.0)


def _tcC(sp, g, dis, b2d):
    return pl.pallas_call(
        _tcC_body,
        grid=(N // BLK,),
        in_specs=[
            pl.BlockSpec((NC, BLK, DF), lambda i: (0, i, 0)),
            pl.BlockSpec((BLK, DF), lambda i: (i, 0)),
            pl.BlockSpec((BLK, 1), lambda i: (i, 0)),
            pl.BlockSpec((1, DF), lambda i: (0, 0)),
        ],
        out_specs=pl.BlockSpec((BLK, DF), lambda i: (i, 0)),
        out_shape=jax.ShapeDtypeStruct((N, DF), jnp.float32),
    )(sp, g, dis, b2d)


def _tcD_body(sp_ref, u_ref, dis_ref, w_ref, b_ref, batch_ref, out_ref):
    t = sp_ref[0] + sp_ref[1] + u_ref[...]
    h = dis_ref[...] * jnp.dot(t, w_ref[...], preferred_element_type=jnp.float32,
                               precision=_HI) + b_ref[...]
    ids = lax.broadcasted_iota(jnp.int32, (NG, N), 0)
    m = (ids == batch_ref[...]).astype(jnp.float32)
    sums = jnp.dot(m, h, preferred_element_type=jnp.float32, precision=_HI)
    counts = jnp.sum(m, axis=1, keepdims=True)
    pooled = sums / jnp.maximum(counts, 1.0)
    logits = pooled[:, :NCLS]
    mx = jnp.max(logits, axis=1, keepdims=True)
    s = logits - mx
    lse = jnp.log(jnp.sum(jnp.exp(s), axis=1, keepdims=True))
    out_ref[...] = s - lse


def _tcD(sp, u, dis, W3p, b2d, batch2d):
    return pl.pallas_call(
        _tcD_body,
        in_specs=[
            pl.BlockSpec((NC, N, DF), lambda: (0, 0, 0)),
            pl.BlockSpec((N, DF), lambda: (0, 0)),
            pl.BlockSpec((N, 1), lambda: (0, 0)),
            pl.BlockSpec((DF, DP), lambda: (0, 0)),
            pl.BlockSpec((1, DP), lambda: (0, 0)),
            pl.BlockSpec((1, N), lambda: (0, 0)),
        ],
        out_specs=pl.BlockSpec((NG, NCLS), lambda: (0, 0)),
        out_shape=jax.ShapeDtypeStruct((NG, NCLS), jnp.float32),
    )(sp, u, dis, W3p, b2d, batch2d)


def kernel(x, edge_index, batch, W1, b1, W2, b2, W3, b3):
    src3 = edge_index[0]
    dst3 = edge_index[1]
    z128 = jnp.zeros((N, DF), jnp.float32)

    degp = _make_deg()(edge_index[1], jnp.zeros((NPAD,), jnp.float32))
    dis = _tcdis(degp)
    g1 = _tcA(x, W1, dis)
    s1 = _scat(g1, src3, dst3, z128)
    g2 = _tcB(s1, g1, dis, b1.reshape(1, DF), W2)
    s2 = _scat(g2, src3, dst3, z128)
    u = _tcC(s2, g2, dis, b2.reshape(1, DF))
    s3 = _scat(u, src3, dst3, z128)
    W3p = jnp.zeros((DF, DP), jnp.float32).at[:, :NCLS].set(W3)
    b3p = jnp.zeros((1, DP), jnp.float32).at[:, :NCLS].set(b3.reshape(1, NCLS))
    return _tcD(s3, u, dis, W3p, b3p, batch.reshape(1, N))
